# baseline (speedup 1.0000x reference)
import functools

import jax
import jax.numpy as jnp
from jax import lax
from jax.experimental import pallas as pl
from jax.experimental.pallas import tpu as pltpu
from jax.experimental.pallas import tpu_sc as plsc

B, S, D0, D1, D2 = 1024, 20, 256, 128, 64
M, K, A = 65536, 32, 16
GS = 128
NG = M // GS
TOPG = 40
NCAND = TOPG * GS
BB5 = 256

BB = 128
MB = 2048

SC_CORES, SC_SUBCORES = 2, 16
NW = SC_CORES * SC_SUBCORES

_NEG = -3.0e38
_IBIG = 2**31 - 1


def _encode_kernel(x_ref, we_ref, be_ref, w1_ref, b1_ref, w2_ref, b2_ref,
                   proto_ref, s0_ref, qn_ref, s1_ref, s2_ref, bel_ref,
                   ba_ref, befe_ref, fe_ref, acc_ref):
    i = pl.program_id(0)
    pooled = jnp.mean(x_ref[...], axis=1)
    s0 = jnp.tanh(jnp.dot(pooled, we_ref[...]) + be_ref[...])
    s1 = jnp.tanh(jnp.dot(s0, w1_ref[...]) + b1_ref[...])
    s2 = jnp.tanh(jnp.dot(s1, w2_ref[...]) + b2_ref[...])
    s0_ref[...] = s0
    s1_ref[...] = s1
    s2_ref[...] = s2
    qn_ref[...] = s0 / (jnp.sqrt(jnp.sum(s0 * s0, axis=1, keepdims=True)) + 1e-6)

    @pl.when(i == 0)
    def _():
        acc_ref[0] = 0.0
        acc_ref[1] = 0.0
        acc_ref[2] = 0.0

    acc_ref[0] += jnp.sum(s0 * s0)
    acc_ref[1] += jnp.sum(s1 * s1)
    acc_ref[2] += jnp.sum(s2 * s2)

    @pl.when(i == pl.num_programs(0) - 1)
    def _():
        fe = 0.5 * (acc_ref[0] / (B * D0) + acc_ref[1] / (B * D1)
                    + acc_ref[2] / (B * D2))
        fe_ref[...] = jnp.broadcast_to(fe, (1, 1))

    d = s2[:, None, :] - proto_ref[...][None, :, :]
    efe = jnp.sum(d * d, axis=-1)
    mx = jnp.max(-efe, axis=-1, keepdims=True)
    e = jnp.exp(-efe - mx)
    bel_ref[...] = e / jnp.sum(e, axis=-1, keepdims=True)
    mn = jnp.min(efe, axis=-1, keepdims=True)
    befe_ref[...] = mn
    ai = lax.broadcasted_iota(jnp.int32, efe.shape, 1)
    ba_ref[...] = jnp.min(jnp.where(efe == mn, ai, _IBIG), axis=-1,
                          keepdims=True)


def _encode(x, W_enc0, b_enc0, W_l1, b_l1, W_l2, b_l2, action_proto):
    grid = B // BB
    full = lambda shape: pl.BlockSpec(shape, lambda i: tuple(0 for _ in shape))
    out = pl.pallas_call(
        _encode_kernel,
        grid=(grid,),
        in_specs=[
            pl.BlockSpec((BB, S, D0), lambda i: (i, 0, 0)),
            full((D0, D0)), full((1, D0)),
            full((D0, D1)), full((1, D1)),
            full((D1, D2)), full((1, D2)),
            full((A, D2)),
        ],
        out_specs=[
            pl.BlockSpec((BB, D0), lambda i: (i, 0)),
            pl.BlockSpec((BB, D0), lambda i: (i, 0)),
            pl.BlockSpec((BB, D1), lambda i: (i, 0)),
            pl.BlockSpec((BB, D2), lambda i: (i, 0)),
            pl.BlockSpec((BB, A), lambda i: (i, 0)),
            pl.BlockSpec((BB, 1), lambda i: (i, 0)),
            pl.BlockSpec((BB, 1), lambda i: (i, 0)),
            pl.BlockSpec((1, 1), lambda i: (0, 0)),
        ],
        out_shape=[
            jax.ShapeDtypeStruct((B, D0), jnp.float32),
            jax.ShapeDtypeStruct((B, D0), jnp.float32),
            jax.ShapeDtypeStruct((B, D1), jnp.float32),
            jax.ShapeDtypeStruct((B, D2), jnp.float32),
            jax.ShapeDtypeStruct((B, A), jnp.float32),
            jax.ShapeDtypeStruct((B, 1), jnp.int32),
            jax.ShapeDtypeStruct((B, 1), jnp.float32),
            jax.ShapeDtypeStruct((1, 1), jnp.float32),
        ],
        scratch_shapes=[pltpu.SMEM((3,), jnp.float32)],
    )(x, W_enc0, b_enc0.reshape(1, D0), W_l1, b_l1.reshape(1, D1),
      W_l2, b_l2.reshape(1, D2), action_proto)
    return out


def _sims_kernel(qn_ref, mem_ref, sims_ref, gmax_ref):
    j = pl.program_id(0)
    mb = mem_ref[...]
    inv = 1.0 / (jnp.sqrt(jnp.sum(mb * mb, axis=1, keepdims=True)) + 1e-6)
    mn = mb * inv
    s = lax.dot_general(qn_ref[...], mn, (((1,), (1,)), ((), ())))
    sims_ref[...] = s
    g = jnp.max(s.reshape(B, MB // GS, GS), axis=2)
    gg = MB // GS
    ri = lax.broadcasted_iota(jnp.int32, (gg, NG), 0)
    ci = lax.broadcasted_iota(jnp.int32, (gg, NG), 1)
    sel = (ci == ri + j * gg).astype(jnp.float32)
    placed = jnp.dot(g, sel, precision=lax.Precision.HIGHEST)

    @pl.when(j == 0)
    def _():
        gmax_ref[...] = jnp.zeros((B, NG), jnp.float32)

    gmax_ref[...] += placed


def _sims(qn, mem):
    grid = M // MB
    return pl.pallas_call(
        _sims_kernel,
        grid=(grid,),
        in_specs=[
            pl.BlockSpec((B, D0), lambda j: (0, 0)),
            pl.BlockSpec((MB, D0), lambda j: (j, 0)),
        ],
        out_specs=[
            pl.BlockSpec((B, MB), lambda j: (0, j)),
            pl.BlockSpec((B, NG), lambda j: (0, 0)),
        ],
        out_shape=[
            jax.ShapeDtypeStruct((B, M), jnp.float32),
            jax.ShapeDtypeStruct((B, NG), jnp.float32),
        ],
    )(qn, mem)


def _topg_kernel(gmax_ref, out_ref, w_ref):
    w_ref[...] = gmax_ref[...]
    gi = lax.broadcasted_iota(jnp.int32, (B, NG), 1)
    bi = lax.broadcasted_iota(jnp.int32, (B, 64), 0)
    li = lax.broadcasted_iota(jnp.int32, (B, 64), 1)
    out_ref[...] = jnp.zeros((B, 64), jnp.int32)

    def body(g, _):
        w = w_ref[...]
        m = jnp.max(w, axis=1, keepdims=True)
        idx = jnp.min(jnp.where(w == m, gi, _IBIG), axis=1, keepdims=True)
        w_ref[...] = jnp.where(gi == idx, _NEG, w)
        out_ref[...] += jnp.where(li == g, idx + bi * NG, 0)
        return 0

    lax.fori_loop(0, TOPG, body, 0)


def _topg(gmax):
    return pl.pallas_call(
        _topg_kernel,
        out_shape=jax.ShapeDtypeStruct((B, 64), jnp.int32),
        scratch_shapes=[pltpu.VMEM((B, NG), jnp.float32)],
    )(gmax)


def _sc_gather(table, idx, rows_per_chunk):
    n, d = len(idx), table.shape[1]
    per_w = n // NW
    nchunk = per_w // rows_per_chunk
    assert per_w % rows_per_chunk == 0 and per_w % 8 == 0
    mesh = plsc.VectorSubcoreMesh(core_axis_name="c", subcore_axis_name="s")

    @functools.partial(
        pl.kernel,
        out_type=jax.ShapeDtypeStruct((n, d), jnp.float32),
        mesh=mesh,
        scratch_types=[
            pltpu.VMEM((per_w,), jnp.int32),
            pltpu.VMEM((rows_per_chunk, d), jnp.float32),
            pltpu.SemaphoreType.DMA,
        ],
    )
    def k(table_hbm, idx_hbm, out_hbm, idx_v, rows_v, sem):
        wid = lax.axis_index("s") * SC_CORES + lax.axis_index("c")
        base = wid * per_w
        pltpu.sync_copy(idx_hbm.at[pl.ds(base, per_w)], idx_v)

        def chunk(c, _):
            off = c * rows_per_chunk
            pltpu.async_copy(
                table_hbm.at[idx_v.at[pl.ds(off, rows_per_chunk)]],
                rows_v, sem).wait()
            pltpu.sync_copy(rows_v, out_hbm.at[pl.ds(base + off, rows_per_chunk)])
            return 0

        lax.fori_loop(0, nchunk, chunk, 0)

    return k(table, idx)


def _topk_kernel(cv_ref, tg_ref, out_ref, w_ref):
    i = pl.program_id(0)
    bi = lax.broadcasted_iota(jnp.int32, (BB5, TOPG), 0) + i * BB5
    graw = tg_ref[...] - bi * NG
    cidx = (graw[:, :, None] * GS
            + lax.broadcasted_iota(jnp.int32, (BB5, TOPG, GS), 2))
    li = lax.broadcasted_iota(jnp.int32, (BB5, K), 1)

    w_ref[...] = cv_ref[...]
    vals, idxs = [], []
    for _ in range(8):
        w = w_ref[...]
        m = jnp.max(w, axis=2)
        ix = jnp.min(jnp.where(w == m[:, :, None], cidx, _IBIG), axis=2)
        w_ref[...] = jnp.where(cidx == ix[:, :, None], _NEG, w)
        vals.append(m)
        idxs.append(ix)
    cv2 = jnp.concatenate(vals, axis=1)
    ci2 = jnp.concatenate(idxs, axis=1)

    outv = jnp.zeros((BB5, K), jnp.int32)
    w2 = cv2
    v32 = None
    for t in range(K):
        m = jnp.max(w2, axis=1, keepdims=True)
        ix = jnp.min(jnp.where(w2 == m, ci2, _IBIG), axis=1, keepdims=True)
        w2 = jnp.where(ci2 == ix, _NEG, w2)
        outv += jnp.where(li == t, ix, 0)
        v32 = m
    out_ref[...] = outv

    flag = jnp.any(vals[7] >= v32)

    @pl.when(flag)
    def _():
        w_ref[...] = cv_ref[...]
        out_ref[...] = jnp.zeros((BB5, K), jnp.int32)

        def body(t, _):
            w = w_ref[...]
            m = jnp.max(jnp.max(w, axis=2), axis=1)[:, None, None]
            cand = jnp.where(w == m, cidx, _IBIG)
            idx = jnp.min(jnp.min(cand, axis=2), axis=1)[:, None]
            w_ref[...] = jnp.where(cidx == idx[:, :, None], _NEG, w)
            out_ref[...] += jnp.where(li == t, idx, 0)
            return 0

        lax.fori_loop(0, K, body, 0)


def _topk(candv, topg):
    grid = B // BB5
    return pl.pallas_call(
        _topk_kernel,
        grid=(grid,),
        in_specs=[
            pl.BlockSpec((BB5, TOPG, GS), lambda i: (i, 0, 0)),
            pl.BlockSpec((BB5, TOPG), lambda i: (i, 0)),
        ],
        out_specs=pl.BlockSpec((BB5, K), lambda i: (i, 0)),
        out_shape=jax.ShapeDtypeStruct((B, K), jnp.int32),
        scratch_shapes=[pltpu.VMEM((BB5, TOPG, GS), jnp.float32)],
    )(candv, topg)


def _integrate_kernel(rows_ref, s0_ref, wi_ref, bi_ref, ws0_ref, sym_ref):
    ctx = jnp.sum(rows_ref[...], axis=1) * (1.0 / K)
    comb = jnp.concatenate([s0_ref[...], ctx], axis=1)
    ws0 = jnp.dot(comb, wi_ref[...]) + bi_ref[...]
    ws0_ref[...] = ws0
    sym_ref[...] = jnp.sum((ws0 > 0.5).astype(jnp.int32), axis=1,
                           keepdims=True)


def _integrate(rows, s0, W_int, b_int):
    grid = B // BB
    full = lambda shape: pl.BlockSpec(shape, lambda i: tuple(0 for _ in shape))
    return pl.pallas_call(
        _integrate_kernel,
        grid=(grid,),
        in_specs=[
            pl.BlockSpec((BB, K, D0), lambda i: (i, 0, 0)),
            pl.BlockSpec((BB, D0), lambda i: (i, 0)),
            full((2 * D0, D0)), full((1, D0)),
        ],
        out_specs=[
            pl.BlockSpec((BB, D0), lambda i: (i, 0)),
            pl.BlockSpec((BB, 1), lambda i: (i, 0)),
        ],
        out_shape=[
            jax.ShapeDtypeStruct((B, D0), jnp.float32),
            jax.ShapeDtypeStruct((B, 1), jnp.int32),
        ],
    )(rows, s0, W_int, b_int.reshape(1, D0))


def _winner_kernel(wc_ref, wr_ref, out_ref):
    eq = wc_ref[...] == wr_ref[...]
    ji = lax.broadcasted_iota(jnp.int32, (B, B), 1)
    out_ref[...] = jnp.max(jnp.where(eq, ji, -1), axis=1, keepdims=True)


def _winner(write_idx):
    return pl.pallas_call(
        _winner_kernel,
        out_shape=jax.ShapeDtypeStruct((B, 1), jnp.int32),
    )(write_idx.reshape(B, 1), write_idx.reshape(1, B))


def _scatter_kernel(mem_ref, ws0_ref, widx_ref, win_ref, out_ref, sem):
    def fire(b, _):
        wb = win_ref[b]
        t = widx_ref[b]
        pltpu.make_async_copy(ws0_ref.at[pl.ds(wb, 1), :],
                              out_ref.at[pl.ds(t, 1), :], sem).start()
        return 0

    lax.fori_loop(0, B, fire, 0)

    def drain(b, _):
        pltpu.make_async_copy(ws0_ref.at[pl.ds(0, 1), :],
                              out_ref.at[pl.ds(0, 1), :], sem).wait()
        return 0

    lax.fori_loop(0, B, drain, 0)


def _scatter(mem, ws0, write_idx, winner):
    return pl.pallas_call(
        _scatter_kernel,
        in_specs=[
            pl.BlockSpec(memory_space=pltpu.MemorySpace.HBM),
            pl.BlockSpec((B, D0), lambda: (0, 0)),
            pl.BlockSpec(memory_space=pltpu.MemorySpace.SMEM),
            pl.BlockSpec(memory_space=pltpu.MemorySpace.SMEM),
        ],
        out_specs=pl.BlockSpec(memory_space=pltpu.MemorySpace.HBM),
        out_shape=jax.ShapeDtypeStruct((M, D0), jnp.float32),
        scratch_shapes=[pltpu.SemaphoreType.DMA],
        input_output_aliases={0: 0},
    )(mem, ws0, write_idx, winner)


def kernel(x, mem, write_idx, W_enc0, b_enc0, W_l1, b_l1, W_l2, b_l2,
           W_int, b_int, action_proto):
    s0, qn, s1, s2, belief, ba, befe, fe = _encode(
        x, W_enc0, b_enc0, W_l1, b_l1, W_l2, b_l2, action_proto)

    sims, gmax = _sims(qn, mem)

    topg = _topg(gmax)[:, :TOPG]

    candv = _sc_gather(sims.reshape(B * NG, GS), topg.reshape(B * TOPG),
                       rows_per_chunk=min(640, B * TOPG // NW))
    candv = candv.reshape(B, TOPG, GS)

    topidx = _topk(candv, topg)

    rows = _sc_gather(mem, topidx.reshape(B * K), rows_per_chunk=256)
    rows = rows.reshape(B, K, D0)

    ws0, sym = _integrate(rows, s0, W_int, b_int)

    winner = _winner(write_idx.astype(jnp.int32))
    new_mem = _scatter(mem, ws0, write_idx.astype(jnp.int32),
                       winner.reshape(B))

    return (ws0, s1, s2, fe.reshape(()), ba.reshape(B), belief,
            befe.reshape(B), sym.reshape(B), new_mem)


# speedup vs baseline: 1.0000x; 1.0000x over previous
import functools

import jax
import jax.numpy as jnp
from jax import lax
from jax.experimental import pallas as pl
from jax.experimental.pallas import tpu as pltpu
from jax.experimental.pallas import tpu_sc as plsc

B, S, D0, D1, D2 = 1024, 20, 256, 128, 64
M, K, A = 65536, 32, 16
GS = 128
NG = M // GS
TOPG = 40
NCAND = TOPG * GS
BB5 = 256

BB = 128
MB = 2048

SC_CORES, SC_SUBCORES = 2, 16
NW = SC_CORES * SC_SUBCORES

_NEG = -3.0e38
_IBIG = 2**31 - 1


def _encode_kernel(x_ref, we_ref, be_ref, w1_ref, b1_ref, w2_ref, b2_ref,
                   proto_ref, s0_ref, qn_ref, s1_ref, s2_ref, bel_ref,
                   ba_ref, befe_ref, fe_ref, acc_ref):
    i = pl.program_id(0)
    pooled = jnp.mean(x_ref[...], axis=1)
    s0 = jnp.tanh(jnp.dot(pooled, we_ref[...]) + be_ref[...])
    s1 = jnp.tanh(jnp.dot(s0, w1_ref[...]) + b1_ref[...])
    s2 = jnp.tanh(jnp.dot(s1, w2_ref[...]) + b2_ref[...])
    s0_ref[...] = s0
    s1_ref[...] = s1
    s2_ref[...] = s2
    qn_ref[...] = s0 / (jnp.sqrt(jnp.sum(s0 * s0, axis=1, keepdims=True)) + 1e-6)

    @pl.when(i == 0)
    def _():
        acc_ref[0] = 0.0
        acc_ref[1] = 0.0
        acc_ref[2] = 0.0

    acc_ref[0] += jnp.sum(s0 * s0)
    acc_ref[1] += jnp.sum(s1 * s1)
    acc_ref[2] += jnp.sum(s2 * s2)

    @pl.when(i == pl.num_programs(0) - 1)
    def _():
        fe = 0.5 * (acc_ref[0] / (B * D0) + acc_ref[1] / (B * D1)
                    + acc_ref[2] / (B * D2))
        fe_ref[...] = jnp.broadcast_to(fe, (1, 1))

    d = s2[:, None, :] - proto_ref[...][None, :, :]
    efe = jnp.sum(d * d, axis=-1)
    mx = jnp.max(-efe, axis=-1, keepdims=True)
    e = jnp.exp(-efe - mx)
    bel_ref[...] = e / jnp.sum(e, axis=-1, keepdims=True)
    mn = jnp.min(efe, axis=-1, keepdims=True)
    befe_ref[...] = mn
    ai = lax.broadcasted_iota(jnp.int32, efe.shape, 1)
    ba_ref[...] = jnp.min(jnp.where(efe == mn, ai, _IBIG), axis=-1,
                          keepdims=True)


def _encode(x, W_enc0, b_enc0, W_l1, b_l1, W_l2, b_l2, action_proto):
    grid = B // BB
    full = lambda shape: pl.BlockSpec(shape, lambda i: tuple(0 for _ in shape))
    out = pl.pallas_call(
        _encode_kernel,
        grid=(grid,),
        in_specs=[
            pl.BlockSpec((BB, S, D0), lambda i: (i, 0, 0)),
            full((D0, D0)), full((1, D0)),
            full((D0, D1)), full((1, D1)),
            full((D1, D2)), full((1, D2)),
            full((A, D2)),
        ],
        out_specs=[
            pl.BlockSpec((BB, D0), lambda i: (i, 0)),
            pl.BlockSpec((BB, D0), lambda i: (i, 0)),
            pl.BlockSpec((BB, D1), lambda i: (i, 0)),
            pl.BlockSpec((BB, D2), lambda i: (i, 0)),
            pl.BlockSpec((BB, A), lambda i: (i, 0)),
            pl.BlockSpec((BB, 1), lambda i: (i, 0)),
            pl.BlockSpec((BB, 1), lambda i: (i, 0)),
            pl.BlockSpec((1, 1), lambda i: (0, 0)),
        ],
        out_shape=[
            jax.ShapeDtypeStruct((B, D0), jnp.float32),
            jax.ShapeDtypeStruct((B, D0), jnp.float32),
            jax.ShapeDtypeStruct((B, D1), jnp.float32),
            jax.ShapeDtypeStruct((B, D2), jnp.float32),
            jax.ShapeDtypeStruct((B, A), jnp.float32),
            jax.ShapeDtypeStruct((B, 1), jnp.int32),
            jax.ShapeDtypeStruct((B, 1), jnp.float32),
            jax.ShapeDtypeStruct((1, 1), jnp.float32),
        ],
        scratch_shapes=[pltpu.SMEM((3,), jnp.float32)],
    )(x, W_enc0, b_enc0.reshape(1, D0), W_l1, b_l1.reshape(1, D1),
      W_l2, b_l2.reshape(1, D2), action_proto)
    return out


def _sims_kernel(qn_ref, mem_ref, sims_ref, gmax_ref):
    j = pl.program_id(0)
    mb = mem_ref[...]
    mn = mb / (jnp.sqrt(jnp.sum(mb * mb, axis=1, keepdims=True)) + 1e-6)
    s = lax.dot_general(qn_ref[...], mn, (((1,), (1,)), ((), ())))
    sims_ref[...] = s
    g = jnp.max(s.reshape(B, MB // GS, GS), axis=2)
    gg = MB // GS
    ri = lax.broadcasted_iota(jnp.int32, (gg, NG), 0)
    ci = lax.broadcasted_iota(jnp.int32, (gg, NG), 1)
    sel = (ci == ri + j * gg).astype(jnp.float32)
    placed = jnp.dot(g, sel, precision=lax.Precision.HIGHEST)

    @pl.when(j == 0)
    def _():
        gmax_ref[...] = jnp.zeros((B, NG), jnp.float32)

    gmax_ref[...] += placed


def _sims(qn, mem):
    grid = M // MB
    return pl.pallas_call(
        _sims_kernel,
        grid=(grid,),
        in_specs=[
            pl.BlockSpec((B, D0), lambda j: (0, 0)),
            pl.BlockSpec((MB, D0), lambda j: (j, 0)),
        ],
        out_specs=[
            pl.BlockSpec((B, MB), lambda j: (0, j)),
            pl.BlockSpec((B, NG), lambda j: (0, 0)),
        ],
        out_shape=[
            jax.ShapeDtypeStruct((B, M), jnp.float32),
            jax.ShapeDtypeStruct((B, NG), jnp.float32),
        ],
    )(qn, mem)


def _topg_kernel(gmax_ref, out_ref, w_ref):
    w_ref[...] = gmax_ref[...]
    gi = lax.broadcasted_iota(jnp.int32, (B, NG), 1)
    bi = lax.broadcasted_iota(jnp.int32, (B, 64), 0)
    li = lax.broadcasted_iota(jnp.int32, (B, 64), 1)
    out_ref[...] = jnp.zeros((B, 64), jnp.int32)

    def body(g, _):
        w = w_ref[...]
        m = jnp.max(w, axis=1, keepdims=True)
        idx = jnp.min(jnp.where(w == m, gi, _IBIG), axis=1, keepdims=True)
        w_ref[...] = jnp.where(gi == idx, _NEG, w)
        out_ref[...] += jnp.where(li == g, idx + bi * NG, 0)
        return 0

    lax.fori_loop(0, TOPG, body, 0)


def _topg(gmax):
    return pl.pallas_call(
        _topg_kernel,
        out_shape=jax.ShapeDtypeStruct((B, 64), jnp.int32),
        scratch_shapes=[pltpu.VMEM((B, NG), jnp.float32)],
    )(gmax)


def _sc_gather(table, idx, rows_per_chunk):
    n, d = len(idx), table.shape[1]
    per_w = n // NW
    nchunk = per_w // rows_per_chunk
    assert per_w % rows_per_chunk == 0 and per_w % 8 == 0
    mesh = plsc.VectorSubcoreMesh(core_axis_name="c", subcore_axis_name="s")

    @functools.partial(
        pl.kernel,
        out_type=jax.ShapeDtypeStruct((n, d), jnp.float32),
        mesh=mesh,
        scratch_types=[
            pltpu.VMEM((per_w,), jnp.int32),
            pltpu.VMEM((rows_per_chunk, d), jnp.float32),
            pltpu.SemaphoreType.DMA,
        ],
    )
    def k(table_hbm, idx_hbm, out_hbm, idx_v, rows_v, sem):
        wid = lax.axis_index("s") * SC_CORES + lax.axis_index("c")
        base = wid * per_w
        pltpu.sync_copy(idx_hbm.at[pl.ds(base, per_w)], idx_v)

        def chunk(c, _):
            off = c * rows_per_chunk
            pltpu.async_copy(
                table_hbm.at[idx_v.at[pl.ds(off, rows_per_chunk)]],
                rows_v, sem).wait()
            pltpu.sync_copy(rows_v, out_hbm.at[pl.ds(base + off, rows_per_chunk)])
            return 0

        lax.fori_loop(0, nchunk, chunk, 0)

    return k(table, idx)


def _topk_kernel(cv_ref, tg_ref, out_ref, w_ref):
    i = pl.program_id(0)
    bi = lax.broadcasted_iota(jnp.int32, (BB5, TOPG), 0) + i * BB5
    graw = tg_ref[...] - bi * NG
    cidx = (graw[:, :, None] * GS
            + lax.broadcasted_iota(jnp.int32, (BB5, TOPG, GS), 2))
    li = lax.broadcasted_iota(jnp.int32, (BB5, K), 1)

    w_ref[...] = cv_ref[...]
    vals, idxs = [], []
    for _ in range(8):
        w = w_ref[...]
        m = jnp.max(w, axis=2)
        ix = jnp.min(jnp.where(w == m[:, :, None], cidx, _IBIG), axis=2)
        w_ref[...] = jnp.where(cidx == ix[:, :, None], _NEG, w)
        vals.append(m)
        idxs.append(ix)
    cv2 = jnp.concatenate(vals, axis=1)
    ci2 = jnp.concatenate(idxs, axis=1)

    outv = jnp.zeros((BB5, K), jnp.int32)
    w2 = cv2
    v32 = None
    for t in range(K):
        m = jnp.max(w2, axis=1, keepdims=True)
        ix = jnp.min(jnp.where(w2 == m, ci2, _IBIG), axis=1, keepdims=True)
        w2 = jnp.where(ci2 == ix, _NEG, w2)
        outv += jnp.where(li == t, ix, 0)
        v32 = m
    out_ref[...] = outv

    flag = jnp.any(vals[7] >= v32)

    @pl.when(flag)
    def _():
        w_ref[...] = cv_ref[...]
        out_ref[...] = jnp.zeros((BB5, K), jnp.int32)

        def body(t, _):
            w = w_ref[...]
            m = jnp.max(jnp.max(w, axis=2), axis=1)[:, None, None]
            cand = jnp.where(w == m, cidx, _IBIG)
            idx = jnp.min(jnp.min(cand, axis=2), axis=1)[:, None]
            w_ref[...] = jnp.where(cidx == idx[:, :, None], _NEG, w)
            out_ref[...] += jnp.where(li == t, idx, 0)
            return 0

        lax.fori_loop(0, K, body, 0)


def _topk(candv, topg):
    grid = B // BB5
    return pl.pallas_call(
        _topk_kernel,
        grid=(grid,),
        in_specs=[
            pl.BlockSpec((BB5, TOPG, GS), lambda i: (i, 0, 0)),
            pl.BlockSpec((BB5, TOPG), lambda i: (i, 0)),
        ],
        out_specs=pl.BlockSpec((BB5, K), lambda i: (i, 0)),
        out_shape=jax.ShapeDtypeStruct((B, K), jnp.int32),
        scratch_shapes=[pltpu.VMEM((BB5, TOPG, GS), jnp.float32)],
    )(candv, topg)


def _integrate_kernel(rows_ref, s0_ref, wi_ref, bi_ref, ws0_ref, sym_ref):
    ctx = jnp.sum(rows_ref[...], axis=1) * (1.0 / K)
    comb = jnp.concatenate([s0_ref[...], ctx], axis=1)
    ws0 = jnp.dot(comb, wi_ref[...]) + bi_ref[...]
    ws0_ref[...] = ws0
    sym_ref[...] = jnp.sum((ws0 > 0.5).astype(jnp.int32), axis=1,
                           keepdims=True)


def _integrate(rows, s0, W_int, b_int):
    grid = B // BB
    full = lambda shape: pl.BlockSpec(shape, lambda i: tuple(0 for _ in shape))
    return pl.pallas_call(
        _integrate_kernel,
        grid=(grid,),
        in_specs=[
            pl.BlockSpec((BB, K, D0), lambda i: (i, 0, 0)),
            pl.BlockSpec((BB, D0), lambda i: (i, 0)),
            full((2 * D0, D0)), full((1, D0)),
        ],
        out_specs=[
            pl.BlockSpec((BB, D0), lambda i: (i, 0)),
            pl.BlockSpec((BB, 1), lambda i: (i, 0)),
        ],
        out_shape=[
            jax.ShapeDtypeStruct((B, D0), jnp.float32),
            jax.ShapeDtypeStruct((B, 1), jnp.int32),
        ],
    )(rows, s0, W_int, b_int.reshape(1, D0))


def _winner_kernel(wc_ref, wr_ref, out_ref):
    eq = wc_ref[...] == wr_ref[...]
    ji = lax.broadcasted_iota(jnp.int32, (B, B), 1)
    out_ref[...] = jnp.max(jnp.where(eq, ji, -1), axis=1, keepdims=True)


def _winner(write_idx):
    return pl.pallas_call(
        _winner_kernel,
        out_shape=jax.ShapeDtypeStruct((B, 1), jnp.int32),
    )(write_idx.reshape(B, 1), write_idx.reshape(1, B))


def _scatter_kernel(mem_ref, ws0_ref, widx_ref, win_ref, out_ref, sem):
    def fire(b, _):
        wb = win_ref[b]
        t = widx_ref[b]
        pltpu.make_async_copy(ws0_ref.at[pl.ds(wb, 1), :],
                              out_ref.at[pl.ds(t, 1), :], sem).start()
        return 0

    lax.fori_loop(0, B, fire, 0)

    def drain(b, _):
        pltpu.make_async_copy(ws0_ref.at[pl.ds(0, 1), :],
                              out_ref.at[pl.ds(0, 1), :], sem).wait()
        return 0

    lax.fori_loop(0, B, drain, 0)


def _scatter(mem, ws0, write_idx, winner):
    return pl.pallas_call(
        _scatter_kernel,
        in_specs=[
            pl.BlockSpec(memory_space=pltpu.MemorySpace.HBM),
            pl.BlockSpec((B, D0), lambda: (0, 0)),
            pl.BlockSpec(memory_space=pltpu.MemorySpace.SMEM),
            pl.BlockSpec(memory_space=pltpu.MemorySpace.SMEM),
        ],
        out_specs=pl.BlockSpec(memory_space=pltpu.MemorySpace.HBM),
        out_shape=jax.ShapeDtypeStruct((M, D0), jnp.float32),
        scratch_shapes=[pltpu.SemaphoreType.DMA],
        input_output_aliases={0: 0},
    )(mem, ws0, write_idx, winner)


def kernel(x, mem, write_idx, W_enc0, b_enc0, W_l1, b_l1, W_l2, b_l2,
           W_int, b_int, action_proto):
    s0, qn, s1, s2, belief, ba, befe, fe = _encode(
        x, W_enc0, b_enc0, W_l1, b_l1, W_l2, b_l2, action_proto)

    sims, gmax = _sims(qn, mem)

    topg = _topg(gmax)[:, :TOPG]

    candv = _sc_gather(sims.reshape(B * NG, GS), topg.reshape(B * TOPG),
                       rows_per_chunk=min(640, B * TOPG // NW))
    candv = candv.reshape(B, TOPG, GS)

    topidx = _topk(candv, topg)

    rows = _sc_gather(mem, topidx.reshape(B * K), rows_per_chunk=256)
    rows = rows.reshape(B, K, D0)

    ws0, sym = _integrate(rows, s0, W_int, b_int)

    winner = _winner(write_idx.astype(jnp.int32))
    new_mem = _scatter(mem, ws0, write_idx.astype(jnp.int32),
                       winner.reshape(B))

    return (ws0, s1, s2, fe.reshape(()), ba.reshape(B), belief,
            befe.reshape(B), sym.reshape(B), new_mem)


# speedup vs baseline: 1.0393x; 1.0393x over previous
import functools

import jax
import jax.numpy as jnp
from jax import lax
from jax.experimental import pallas as pl
from jax.experimental.pallas import tpu as pltpu
from jax.experimental.pallas import tpu_sc as plsc

B, S, D0, D1, D2 = 1024, 20, 256, 128, 64
M, K, A = 65536, 32, 16
GS = 128
NG = M // GS
TOPG = 40
NCAND = TOPG * GS
BB5 = 256

BB = 128
MB = 2048

SC_CORES, SC_SUBCORES = 2, 16
NW = SC_CORES * SC_SUBCORES

_NEG = -3.0e38
_IBIG = 2**31 - 1


def _encode_kernel(x_ref, we_ref, be_ref, w1_ref, b1_ref, w2_ref, b2_ref,
                   proto_ref, s0_ref, qn_ref, s1_ref, s2_ref, bel_ref,
                   ba_ref, befe_ref, fe_ref, acc_ref):
    i = pl.program_id(0)
    pooled = jnp.mean(x_ref[...], axis=1)
    s0 = jnp.tanh(jnp.dot(pooled, we_ref[...]) + be_ref[...])
    s1 = jnp.tanh(jnp.dot(s0, w1_ref[...]) + b1_ref[...])
    s2 = jnp.tanh(jnp.dot(s1, w2_ref[...]) + b2_ref[...])
    s0_ref[...] = s0
    s1_ref[...] = s1
    s2_ref[...] = s2
    qn_ref[...] = s0 / (jnp.sqrt(jnp.sum(s0 * s0, axis=1, keepdims=True)) + 1e-6)

    @pl.when(i == 0)
    def _():
        acc_ref[0] = 0.0
        acc_ref[1] = 0.0
        acc_ref[2] = 0.0

    acc_ref[0] += jnp.sum(s0 * s0)
    acc_ref[1] += jnp.sum(s1 * s1)
    acc_ref[2] += jnp.sum(s2 * s2)

    @pl.when(i == pl.num_programs(0) - 1)
    def _():
        fe = 0.5 * (acc_ref[0] / (B * D0) + acc_ref[1] / (B * D1)
                    + acc_ref[2] / (B * D2))
        fe_ref[...] = jnp.broadcast_to(fe, (1, 1))

    d = s2[:, None, :] - proto_ref[...][None, :, :]
    efe = jnp.sum(d * d, axis=-1)
    mx = jnp.max(-efe, axis=-1, keepdims=True)
    e = jnp.exp(-efe - mx)
    bel_ref[...] = e / jnp.sum(e, axis=-1, keepdims=True)
    mn = jnp.min(efe, axis=-1, keepdims=True)
    befe_ref[...] = mn
    ai = lax.broadcasted_iota(jnp.int32, efe.shape, 1)
    ba_ref[...] = jnp.min(jnp.where(efe == mn, ai, _IBIG), axis=-1,
                          keepdims=True)


def _encode(x, W_enc0, b_enc0, W_l1, b_l1, W_l2, b_l2, action_proto):
    grid = B // BB
    full = lambda shape: pl.BlockSpec(shape, lambda i: tuple(0 for _ in shape))
    out = pl.pallas_call(
        _encode_kernel,
        grid=(grid,),
        in_specs=[
            pl.BlockSpec((BB, S, D0), lambda i: (i, 0, 0)),
            full((D0, D0)), full((1, D0)),
            full((D0, D1)), full((1, D1)),
            full((D1, D2)), full((1, D2)),
            full((A, D2)),
        ],
        out_specs=[
            pl.BlockSpec((BB, D0), lambda i: (i, 0)),
            pl.BlockSpec((BB, D0), lambda i: (i, 0)),
            pl.BlockSpec((BB, D1), lambda i: (i, 0)),
            pl.BlockSpec((BB, D2), lambda i: (i, 0)),
            pl.BlockSpec((BB, A), lambda i: (i, 0)),
            pl.BlockSpec((BB, 1), lambda i: (i, 0)),
            pl.BlockSpec((BB, 1), lambda i: (i, 0)),
            pl.BlockSpec((1, 1), lambda i: (0, 0)),
        ],
        out_shape=[
            jax.ShapeDtypeStruct((B, D0), jnp.float32),
            jax.ShapeDtypeStruct((B, D0), jnp.float32),
            jax.ShapeDtypeStruct((B, D1), jnp.float32),
            jax.ShapeDtypeStruct((B, D2), jnp.float32),
            jax.ShapeDtypeStruct((B, A), jnp.float32),
            jax.ShapeDtypeStruct((B, 1), jnp.int32),
            jax.ShapeDtypeStruct((B, 1), jnp.float32),
            jax.ShapeDtypeStruct((1, 1), jnp.float32),
        ],
        scratch_shapes=[pltpu.SMEM((3,), jnp.float32)],
    )(x, W_enc0, b_enc0.reshape(1, D0), W_l1, b_l1.reshape(1, D1),
      W_l2, b_l2.reshape(1, D2), action_proto)
    return out


def _sims_kernel(qn_ref, mem_ref, sims_ref, gmax_ref, mcopy_ref):
    j = pl.program_id(0)
    mb = mem_ref[...]
    mcopy_ref[...] = mb
    mn = mb / (jnp.sqrt(jnp.sum(mb * mb, axis=1, keepdims=True)) + 1e-6)
    s = lax.dot_general(qn_ref[...], mn, (((1,), (1,)), ((), ())))
    sims_ref[...] = s
    g = jnp.max(s.reshape(B, MB // GS, GS), axis=2)
    gg = MB // GS
    ri = lax.broadcasted_iota(jnp.int32, (gg, NG), 0)
    ci = lax.broadcasted_iota(jnp.int32, (gg, NG), 1)
    sel = (ci == ri + j * gg).astype(jnp.float32)
    placed = jnp.dot(g, sel, precision=lax.Precision.HIGHEST)

    @pl.when(j == 0)
    def _():
        gmax_ref[...] = jnp.zeros((B, NG), jnp.float32)

    gmax_ref[...] += placed


def _sims(qn, mem):
    grid = M // MB
    return pl.pallas_call(
        _sims_kernel,
        grid=(grid,),
        in_specs=[
            pl.BlockSpec((B, D0), lambda j: (0, 0)),
            pl.BlockSpec((MB, D0), lambda j: (j, 0)),
        ],
        out_specs=[
            pl.BlockSpec((B, MB), lambda j: (0, j)),
            pl.BlockSpec((B, NG), lambda j: (0, 0)),
            pl.BlockSpec((MB, D0), lambda j: (j, 0)),
        ],
        out_shape=[
            jax.ShapeDtypeStruct((B, M), jnp.float32),
            jax.ShapeDtypeStruct((B, NG), jnp.float32),
            jax.ShapeDtypeStruct((M, D0), jnp.float32),
        ],
    )(qn, mem)


def _topg_kernel(gmax_ref, out_ref, w_ref):
    w_ref[...] = gmax_ref[...]
    gi = lax.broadcasted_iota(jnp.int32, (B, NG), 1)
    bi = lax.broadcasted_iota(jnp.int32, (B, 64), 0)
    li = lax.broadcasted_iota(jnp.int32, (B, 64), 1)
    out_ref[...] = jnp.zeros((B, 64), jnp.int32)

    def body(g, _):
        w = w_ref[...]
        m = jnp.max(w, axis=1, keepdims=True)
        idx = jnp.min(jnp.where(w == m, gi, _IBIG), axis=1, keepdims=True)
        w_ref[...] = jnp.where(gi == idx, _NEG, w)
        out_ref[...] += jnp.where(li == g, idx + bi * NG, 0)
        return 0

    lax.fori_loop(0, TOPG, body, 0)


def _topg(gmax):
    return pl.pallas_call(
        _topg_kernel,
        out_shape=jax.ShapeDtypeStruct((B, 64), jnp.int32),
        scratch_shapes=[pltpu.VMEM((B, NG), jnp.float32)],
    )(gmax)


def _sc_gather(table, idx, rows_per_chunk):
    n, d = len(idx), table.shape[1]
    per_w = n // NW
    nchunk = per_w // rows_per_chunk
    assert per_w % rows_per_chunk == 0 and per_w % 8 == 0
    mesh = plsc.VectorSubcoreMesh(core_axis_name="c", subcore_axis_name="s")

    @functools.partial(
        pl.kernel,
        out_type=jax.ShapeDtypeStruct((n, d), jnp.float32),
        mesh=mesh,
        scratch_types=[
            pltpu.VMEM((per_w,), jnp.int32),
            pltpu.VMEM((rows_per_chunk, d), jnp.float32),
            pltpu.SemaphoreType.DMA,
        ],
    )
    def k(table_hbm, idx_hbm, out_hbm, idx_v, rows_v, sem):
        wid = lax.axis_index("s") * SC_CORES + lax.axis_index("c")
        base = wid * per_w
        pltpu.sync_copy(idx_hbm.at[pl.ds(base, per_w)], idx_v)

        def chunk(c, _):
            off = c * rows_per_chunk
            pltpu.async_copy(
                table_hbm.at[idx_v.at[pl.ds(off, rows_per_chunk)]],
                rows_v, sem).wait()
            pltpu.sync_copy(rows_v, out_hbm.at[pl.ds(base + off, rows_per_chunk)])
            return 0

        lax.fori_loop(0, nchunk, chunk, 0)

    return k(table, idx)


def _topk_kernel(cv_ref, tg_ref, out_ref, w_ref):
    i = pl.program_id(0)
    bi = lax.broadcasted_iota(jnp.int32, (BB5, TOPG), 0) + i * BB5
    graw = tg_ref[...] - bi * NG
    cidx = (graw[:, :, None] * GS
            + lax.broadcasted_iota(jnp.int32, (BB5, TOPG, GS), 2))
    li = lax.broadcasted_iota(jnp.int32, (BB5, K), 1)

    w_ref[...] = cv_ref[...]
    vals, idxs = [], []
    for _ in range(8):
        w = w_ref[...]
        m = jnp.max(w, axis=2)
        ix = jnp.min(jnp.where(w == m[:, :, None], cidx, _IBIG), axis=2)
        w_ref[...] = jnp.where(cidx == ix[:, :, None], _NEG, w)
        vals.append(m)
        idxs.append(ix)
    cv2 = jnp.concatenate(vals, axis=1)
    ci2 = jnp.concatenate(idxs, axis=1)

    outv = jnp.zeros((BB5, K), jnp.int32)
    w2 = cv2
    v32 = None
    for t in range(K):
        m = jnp.max(w2, axis=1, keepdims=True)
        ix = jnp.min(jnp.where(w2 == m, ci2, _IBIG), axis=1, keepdims=True)
        w2 = jnp.where(ci2 == ix, _NEG, w2)
        outv += jnp.where(li == t, ix, 0)
        v32 = m
    out_ref[...] = outv

    flag = jnp.any(vals[7] >= v32)

    @pl.when(flag)
    def _():
        w_ref[...] = cv_ref[...]
        out_ref[...] = jnp.zeros((BB5, K), jnp.int32)

        def body(t, _):
            w = w_ref[...]
            m = jnp.max(jnp.max(w, axis=2), axis=1)[:, None, None]
            cand = jnp.where(w == m, cidx, _IBIG)
            idx = jnp.min(jnp.min(cand, axis=2), axis=1)[:, None]
            w_ref[...] = jnp.where(cidx == idx[:, :, None], _NEG, w)
            out_ref[...] += jnp.where(li == t, idx, 0)
            return 0

        lax.fori_loop(0, K, body, 0)


def _topk(candv, topg):
    grid = B // BB5
    return pl.pallas_call(
        _topk_kernel,
        grid=(grid,),
        in_specs=[
            pl.BlockSpec((BB5, TOPG, GS), lambda i: (i, 0, 0)),
            pl.BlockSpec((BB5, TOPG), lambda i: (i, 0)),
        ],
        out_specs=pl.BlockSpec((BB5, K), lambda i: (i, 0)),
        out_shape=jax.ShapeDtypeStruct((B, K), jnp.int32),
        scratch_shapes=[pltpu.VMEM((BB5, TOPG, GS), jnp.float32)],
    )(candv, topg)


def _integrate_kernel(rows_ref, s0_ref, wi_ref, bi_ref, ws0_ref, sym_ref):
    ctx = jnp.sum(rows_ref[...], axis=1) * (1.0 / K)
    comb = jnp.concatenate([s0_ref[...], ctx], axis=1)
    ws0 = jnp.dot(comb, wi_ref[...]) + bi_ref[...]
    ws0_ref[...] = ws0
    sym_ref[...] = jnp.sum((ws0 > 0.5).astype(jnp.int32), axis=1,
                           keepdims=True)


def _integrate(rows, s0, W_int, b_int):
    grid = B // BB
    full = lambda shape: pl.BlockSpec(shape, lambda i: tuple(0 for _ in shape))
    return pl.pallas_call(
        _integrate_kernel,
        grid=(grid,),
        in_specs=[
            pl.BlockSpec((BB, K, D0), lambda i: (i, 0, 0)),
            pl.BlockSpec((BB, D0), lambda i: (i, 0)),
            full((2 * D0, D0)), full((1, D0)),
        ],
        out_specs=[
            pl.BlockSpec((BB, D0), lambda i: (i, 0)),
            pl.BlockSpec((BB, 1), lambda i: (i, 0)),
        ],
        out_shape=[
            jax.ShapeDtypeStruct((B, D0), jnp.float32),
            jax.ShapeDtypeStruct((B, 1), jnp.int32),
        ],
    )(rows, s0, W_int, b_int.reshape(1, D0))


def _winner_kernel(wc_ref, wr_ref, out_ref):
    eq = wc_ref[...] == wr_ref[...]
    ji = lax.broadcasted_iota(jnp.int32, (B, B), 1)
    out_ref[...] = jnp.max(jnp.where(eq, ji, -1), axis=1, keepdims=True)


def _winner(write_idx):
    return pl.pallas_call(
        _winner_kernel,
        out_shape=jax.ShapeDtypeStruct((B, 1), jnp.int32),
    )(write_idx.reshape(B, 1), write_idx.reshape(1, B))


def _scatter_kernel(mem_ref, ws0_ref, widx_ref, win_ref, out_ref, sem):
    def fire(b, _):
        wb = win_ref[b]
        t = widx_ref[b]
        pltpu.make_async_copy(ws0_ref.at[pl.ds(wb, 1), :],
                              out_ref.at[pl.ds(t, 1), :], sem).start()
        return 0

    lax.fori_loop(0, B, fire, 0)

    def drain(b, _):
        pltpu.make_async_copy(ws0_ref.at[pl.ds(0, 1), :],
                              out_ref.at[pl.ds(0, 1), :], sem).wait()
        return 0

    lax.fori_loop(0, B, drain, 0)


def _scatter(mem, ws0, write_idx, winner):
    return pl.pallas_call(
        _scatter_kernel,
        in_specs=[
            pl.BlockSpec(memory_space=pltpu.MemorySpace.HBM),
            pl.BlockSpec((B, D0), lambda: (0, 0)),
            pl.BlockSpec(memory_space=pltpu.MemorySpace.SMEM),
            pl.BlockSpec(memory_space=pltpu.MemorySpace.SMEM),
        ],
        out_specs=pl.BlockSpec(memory_space=pltpu.MemorySpace.HBM),
        out_shape=jax.ShapeDtypeStruct((M, D0), jnp.float32),
        scratch_shapes=[pltpu.SemaphoreType.DMA],
        input_output_aliases={0: 0},
    )(mem, ws0, write_idx, winner)


def kernel(x, mem, write_idx, W_enc0, b_enc0, W_l1, b_l1, W_l2, b_l2,
           W_int, b_int, action_proto):
    s0, qn, s1, s2, belief, ba, befe, fe = _encode(
        x, W_enc0, b_enc0, W_l1, b_l1, W_l2, b_l2, action_proto)

    sims, gmax, mem0 = _sims(qn, mem)

    topg = _topg(gmax)[:, :TOPG]

    candv = _sc_gather(sims.reshape(B * NG, GS), topg.reshape(B * TOPG),
                       rows_per_chunk=min(640, B * TOPG // NW))
    candv = candv.reshape(B, TOPG, GS)

    topidx = _topk(candv, topg)

    rows = _sc_gather(mem, topidx.reshape(B * K), rows_per_chunk=256)
    rows = rows.reshape(B, K, D0)

    ws0, sym = _integrate(rows, s0, W_int, b_int)

    winner = _winner(write_idx.astype(jnp.int32))
    new_mem = _scatter(mem0, ws0, write_idx.astype(jnp.int32),
                       winner.reshape(B))

    return (ws0, s1, s2, fe.reshape(()), ba.reshape(B), belief,
            befe.reshape(B), sym.reshape(B), new_mem)


# speedup vs baseline: 1.1556x; 1.1118x over previous
import functools

import jax
import jax.numpy as jnp
from jax import lax
from jax.experimental import pallas as pl
from jax.experimental.pallas import tpu as pltpu
from jax.experimental.pallas import tpu_sc as plsc

B, S, D0, D1, D2 = 1024, 20, 256, 128, 64
M, K, A = 65536, 32, 16
GS = 128
NG = M // GS
TOPG = 40
NCAND = TOPG * GS
BB5 = 256

BB = 128
MB = 2048

SC_CORES, SC_SUBCORES = 2, 16
NW = SC_CORES * SC_SUBCORES

_NEG = -3.0e38
_IBIG = 2**31 - 1


def _encode_kernel(x_ref, we_ref, be_ref, w1_ref, b1_ref, w2_ref, b2_ref,
                   proto_ref, s0_ref, qn_ref, s1_ref, s2_ref, bel_ref,
                   ba_ref, befe_ref, fe_ref, acc_ref):
    i = pl.program_id(0)
    pooled = jnp.mean(x_ref[...], axis=1)
    s0 = jnp.tanh(jnp.dot(pooled, we_ref[...]) + be_ref[...])
    s1 = jnp.tanh(jnp.dot(s0, w1_ref[...]) + b1_ref[...])
    s2 = jnp.tanh(jnp.dot(s1, w2_ref[...]) + b2_ref[...])
    s0_ref[...] = s0
    s1_ref[...] = s1
    s2_ref[...] = s2
    qn_ref[...] = s0 / (jnp.sqrt(jnp.sum(s0 * s0, axis=1, keepdims=True)) + 1e-6)

    @pl.when(i == 0)
    def _():
        acc_ref[0] = 0.0
        acc_ref[1] = 0.0
        acc_ref[2] = 0.0

    acc_ref[0] += jnp.sum(s0 * s0)
    acc_ref[1] += jnp.sum(s1 * s1)
    acc_ref[2] += jnp.sum(s2 * s2)

    @pl.when(i == pl.num_programs(0) - 1)
    def _():
        fe = 0.5 * (acc_ref[0] / (B * D0) + acc_ref[1] / (B * D1)
                    + acc_ref[2] / (B * D2))
        fe_ref[...] = jnp.broadcast_to(fe, (1, 1))

    d = s2[:, None, :] - proto_ref[...][None, :, :]
    efe = jnp.sum(d * d, axis=-1)
    mx = jnp.max(-efe, axis=-1, keepdims=True)
    e = jnp.exp(-efe - mx)
    bel_ref[...] = e / jnp.sum(e, axis=-1, keepdims=True)
    mn = jnp.min(efe, axis=-1, keepdims=True)
    befe_ref[...] = mn
    ai = lax.broadcasted_iota(jnp.int32, efe.shape, 1)
    ba_ref[...] = jnp.min(jnp.where(efe == mn, ai, _IBIG), axis=-1,
                          keepdims=True)


def _encode(x, W_enc0, b_enc0, W_l1, b_l1, W_l2, b_l2, action_proto):
    grid = B // BB
    full = lambda shape: pl.BlockSpec(shape, lambda i: tuple(0 for _ in shape))
    out = pl.pallas_call(
        _encode_kernel,
        grid=(grid,),
        in_specs=[
            pl.BlockSpec((BB, S, D0), lambda i: (i, 0, 0)),
            full((D0, D0)), full((1, D0)),
            full((D0, D1)), full((1, D1)),
            full((D1, D2)), full((1, D2)),
            full((A, D2)),
        ],
        out_specs=[
            pl.BlockSpec((BB, D0), lambda i: (i, 0)),
            pl.BlockSpec((BB, D0), lambda i: (i, 0)),
            pl.BlockSpec((BB, D1), lambda i: (i, 0)),
            pl.BlockSpec((BB, D2), lambda i: (i, 0)),
            pl.BlockSpec((BB, A), lambda i: (i, 0)),
            pl.BlockSpec((BB, 1), lambda i: (i, 0)),
            pl.BlockSpec((BB, 1), lambda i: (i, 0)),
            pl.BlockSpec((1, 1), lambda i: (0, 0)),
        ],
        out_shape=[
            jax.ShapeDtypeStruct((B, D0), jnp.float32),
            jax.ShapeDtypeStruct((B, D0), jnp.float32),
            jax.ShapeDtypeStruct((B, D1), jnp.float32),
            jax.ShapeDtypeStruct((B, D2), jnp.float32),
            jax.ShapeDtypeStruct((B, A), jnp.float32),
            jax.ShapeDtypeStruct((B, 1), jnp.int32),
            jax.ShapeDtypeStruct((B, 1), jnp.float32),
            jax.ShapeDtypeStruct((1, 1), jnp.float32),
        ],
        scratch_shapes=[pltpu.SMEM((3,), jnp.float32)],
    )(x, W_enc0, b_enc0.reshape(1, D0), W_l1, b_l1.reshape(1, D1),
      W_l2, b_l2.reshape(1, D2), action_proto)
    return out


def _sims_kernel(qn_ref, mem_ref, sims_ref, gmax_ref, mcopy_ref):
    j = pl.program_id(0)
    mb = mem_ref[...]
    mcopy_ref[...] = mb
    mn = mb / (jnp.sqrt(jnp.sum(mb * mb, axis=1, keepdims=True)) + 1e-6)
    s = lax.dot_general(qn_ref[...], mn, (((1,), (1,)), ((), ())))
    sims_ref[...] = s
    g = jnp.max(s.reshape(B, MB // GS, GS), axis=2)
    gg = MB // GS
    ri = lax.broadcasted_iota(jnp.int32, (gg, NG), 0)
    ci = lax.broadcasted_iota(jnp.int32, (gg, NG), 1)
    sel = (ci == ri + j * gg).astype(jnp.float32)
    placed = jnp.dot(g, sel, precision=lax.Precision.HIGHEST)

    @pl.when(j == 0)
    def _():
        gmax_ref[...] = jnp.zeros((B, NG), jnp.float32)

    gmax_ref[...] += placed


def _sims(qn, mem):
    grid = M // MB
    return pl.pallas_call(
        _sims_kernel,
        grid=(grid,),
        in_specs=[
            pl.BlockSpec((B, D0), lambda j: (0, 0)),
            pl.BlockSpec((MB, D0), lambda j: (j, 0)),
        ],
        out_specs=[
            pl.BlockSpec((B, MB), lambda j: (0, j)),
            pl.BlockSpec((B, NG), lambda j: (0, 0)),
            pl.BlockSpec((MB, D0), lambda j: (j, 0)),
        ],
        out_shape=[
            jax.ShapeDtypeStruct((B, M), jnp.float32),
            jax.ShapeDtypeStruct((B, NG), jnp.float32),
            jax.ShapeDtypeStruct((M, D0), jnp.float32),
        ],
    )(qn, mem)


def _topg_kernel(gmax_ref, out_ref, w_ref):
    w_ref[...] = gmax_ref[...]
    gi = lax.broadcasted_iota(jnp.int32, (B, NG), 1)
    bi = lax.broadcasted_iota(jnp.int32, (B, 64), 0)
    li = lax.broadcasted_iota(jnp.int32, (B, 64), 1)
    out_ref[...] = jnp.zeros((B, 64), jnp.int32)

    def body(g, _):
        w = w_ref[...]
        m = jnp.max(w, axis=1, keepdims=True)
        idx = jnp.min(jnp.where(w == m, gi, _IBIG), axis=1, keepdims=True)
        w_ref[...] = jnp.where(gi == idx, _NEG, w)
        out_ref[...] += jnp.where(li == g, idx + bi * NG, 0)
        return 0

    lax.fori_loop(0, TOPG, body, 0)


def _topg(gmax):
    return pl.pallas_call(
        _topg_kernel,
        out_shape=jax.ShapeDtypeStruct((B, 64), jnp.int32),
        scratch_shapes=[pltpu.VMEM((B, NG), jnp.float32)],
    )(gmax)


def _sc_gather(table, idx, rows_per_chunk):
    n, d = len(idx), table.shape[1]
    per_w = n // NW
    nchunk = per_w // rows_per_chunk
    assert per_w % rows_per_chunk == 0 and per_w % 8 == 0
    mesh = plsc.VectorSubcoreMesh(core_axis_name="c", subcore_axis_name="s")

    @functools.partial(
        pl.kernel,
        out_type=jax.ShapeDtypeStruct((n, d), jnp.float32),
        mesh=mesh,
        scratch_types=[
            pltpu.VMEM((per_w,), jnp.int32),
            pltpu.VMEM((rows_per_chunk, d), jnp.float32),
            pltpu.SemaphoreType.DMA,
        ],
    )
    def k(table_hbm, idx_hbm, out_hbm, idx_v, rows_v, sem):
        wid = lax.axis_index("s") * SC_CORES + lax.axis_index("c")
        base = wid * per_w
        pltpu.sync_copy(idx_hbm.at[pl.ds(base, per_w)], idx_v)

        def chunk(c, _):
            off = c * rows_per_chunk
            pltpu.async_copy(
                table_hbm.at[idx_v.at[pl.ds(off, rows_per_chunk)]],
                rows_v, sem).wait()
            pltpu.sync_copy(rows_v, out_hbm.at[pl.ds(base + off, rows_per_chunk)])
            return 0

        lax.fori_loop(0, nchunk, chunk, 0)

    return k(table, idx)


def _topk_kernel(cv_ref, tg_ref, out_ref, w_ref):
    i = pl.program_id(0)
    bi = lax.broadcasted_iota(jnp.int32, (BB5, TOPG), 0) + i * BB5
    graw = tg_ref[...] - bi * NG
    cidx = (graw[:, :, None] * GS
            + lax.broadcasted_iota(jnp.int32, (BB5, TOPG, GS), 2))
    li = lax.broadcasted_iota(jnp.int32, (BB5, K), 1)

    w_ref[...] = cv_ref[...]
    vals, idxs = [], []
    for _ in range(5):
        w = w_ref[...]
        m = jnp.max(w, axis=2)
        ix = jnp.min(jnp.where(w == m[:, :, None], cidx, _IBIG), axis=2)
        w_ref[...] = jnp.where(cidx == ix[:, :, None], _NEG, w)
        vals.append(m)
        idxs.append(ix)
    cv2 = jnp.concatenate(vals, axis=1)
    ci2 = jnp.concatenate(idxs, axis=1)

    outv = jnp.zeros((BB5, K), jnp.int32)
    w2 = cv2
    v32 = None
    for t in range(K):
        m = jnp.max(w2, axis=1, keepdims=True)
        ix = jnp.min(jnp.where(w2 == m, ci2, _IBIG), axis=1, keepdims=True)
        w2 = jnp.where(ci2 == ix, _NEG, w2)
        outv += jnp.where(li == t, ix, 0)
        v32 = m
    out_ref[...] = outv

    flag = jnp.any(vals[4] >= v32)

    @pl.when(flag)
    def _():
        w_ref[...] = cv_ref[...]
        out_ref[...] = jnp.zeros((BB5, K), jnp.int32)

        def body(t, _):
            w = w_ref[...]
            m = jnp.max(jnp.max(w, axis=2), axis=1)[:, None, None]
            cand = jnp.where(w == m, cidx, _IBIG)
            idx = jnp.min(jnp.min(cand, axis=2), axis=1)[:, None]
            w_ref[...] = jnp.where(cidx == idx[:, :, None], _NEG, w)
            out_ref[...] += jnp.where(li == t, idx, 0)
            return 0

        lax.fori_loop(0, K, body, 0)


def _topk(candv, topg):
    grid = B // BB5
    return pl.pallas_call(
        _topk_kernel,
        grid=(grid,),
        in_specs=[
            pl.BlockSpec((BB5, TOPG, GS), lambda i: (i, 0, 0)),
            pl.BlockSpec((BB5, TOPG), lambda i: (i, 0)),
        ],
        out_specs=pl.BlockSpec((BB5, K), lambda i: (i, 0)),
        out_shape=jax.ShapeDtypeStruct((B, K), jnp.int32),
        scratch_shapes=[pltpu.VMEM((BB5, TOPG, GS), jnp.float32)],
    )(candv, topg)


def _integrate_kernel(rows_ref, s0_ref, wi_ref, bi_ref, ws0_ref, sym_ref):
    ctx = jnp.sum(rows_ref[...], axis=1) * (1.0 / K)
    comb = jnp.concatenate([s0_ref[...], ctx], axis=1)
    ws0 = jnp.dot(comb, wi_ref[...]) + bi_ref[...]
    ws0_ref[...] = ws0
    sym_ref[...] = jnp.sum((ws0 > 0.5).astype(jnp.int32), axis=1,
                           keepdims=True)


def _integrate(rows, s0, W_int, b_int):
    grid = B // BB
    full = lambda shape: pl.BlockSpec(shape, lambda i: tuple(0 for _ in shape))
    return pl.pallas_call(
        _integrate_kernel,
        grid=(grid,),
        in_specs=[
            pl.BlockSpec((BB, K, D0), lambda i: (i, 0, 0)),
            pl.BlockSpec((BB, D0), lambda i: (i, 0)),
            full((2 * D0, D0)), full((1, D0)),
        ],
        out_specs=[
            pl.BlockSpec((BB, D0), lambda i: (i, 0)),
            pl.BlockSpec((BB, 1), lambda i: (i, 0)),
        ],
        out_shape=[
            jax.ShapeDtypeStruct((B, D0), jnp.float32),
            jax.ShapeDtypeStruct((B, 1), jnp.int32),
        ],
    )(rows, s0, W_int, b_int.reshape(1, D0))


def _winner_kernel(wc_ref, wr_ref, out_ref):
    eq = wc_ref[...] == wr_ref[...]
    ji = lax.broadcasted_iota(jnp.int32, (B, B), 1)
    out_ref[...] = jnp.max(jnp.where(eq, ji, -1), axis=1, keepdims=True)


def _winner(write_idx):
    return pl.pallas_call(
        _winner_kernel,
        out_shape=jax.ShapeDtypeStruct((B, 1), jnp.int32),
    )(write_idx.reshape(B, 1), write_idx.reshape(1, B))


def _scatter_kernel(mem_ref, ws0_ref, widx_ref, win_ref, out_ref, sem):
    def fire(b, _):
        wb = win_ref[b]
        t = widx_ref[b]
        pltpu.make_async_copy(ws0_ref.at[pl.ds(wb, 1), :],
                              out_ref.at[pl.ds(t, 1), :], sem).start()
        return 0

    lax.fori_loop(0, B, fire, 0)

    def drain(b, _):
        pltpu.make_async_copy(ws0_ref.at[pl.ds(0, 1), :],
                              out_ref.at[pl.ds(0, 1), :], sem).wait()
        return 0

    lax.fori_loop(0, B, drain, 0)


def _scatter(mem, ws0, write_idx, winner):
    return pl.pallas_call(
        _scatter_kernel,
        in_specs=[
            pl.BlockSpec(memory_space=pltpu.MemorySpace.HBM),
            pl.BlockSpec((B, D0), lambda: (0, 0)),
            pl.BlockSpec(memory_space=pltpu.MemorySpace.SMEM),
            pl.BlockSpec(memory_space=pltpu.MemorySpace.SMEM),
        ],
        out_specs=pl.BlockSpec(memory_space=pltpu.MemorySpace.HBM),
        out_shape=jax.ShapeDtypeStruct((M, D0), jnp.float32),
        scratch_shapes=[pltpu.SemaphoreType.DMA],
        input_output_aliases={0: 0},
    )(mem, ws0, write_idx, winner)


def kernel(x, mem, write_idx, W_enc0, b_enc0, W_l1, b_l1, W_l2, b_l2,
           W_int, b_int, action_proto):
    s0, qn, s1, s2, belief, ba, befe, fe = _encode(
        x, W_enc0, b_enc0, W_l1, b_l1, W_l2, b_l2, action_proto)

    sims, gmax, mem0 = _sims(qn, mem)

    topg = _topg(gmax)[:, :TOPG]

    candv = _sc_gather(sims.reshape(B * NG, GS), topg.reshape(B * TOPG),
                       rows_per_chunk=min(640, B * TOPG // NW))
    candv = candv.reshape(B, TOPG, GS)

    topidx = _topk(candv, topg)

    rows = _sc_gather(mem, topidx.reshape(B * K), rows_per_chunk=256)
    rows = rows.reshape(B, K, D0)

    ws0, sym = _integrate(rows, s0, W_int, b_int)

    winner = _winner(write_idx.astype(jnp.int32))
    new_mem = _scatter(mem0, ws0, write_idx.astype(jnp.int32),
                       winner.reshape(B))

    return (ws0, s1, s2, fe.reshape(()), ba.reshape(B), belief,
            befe.reshape(B), sym.reshape(B), new_mem)
